# baseline (speedup 1.0000x reference)
import functools

import jax
import jax.numpy as jnp
from jax.experimental import pallas as pl
from jax.experimental.pallas import tpu as pltpu

_N = 16384
_C = 1000
_K = 4096
_BLK = 512
_G = _N // _BLK


def _body(labels_ref, x_ref, out_ref, loss_ref):
    i = pl.program_id(0)
    x = x_ref[...]
    lab = labels_ref[0, 0, :]
    m = jnp.max(x, axis=1, keepdims=True)
    s = jnp.sum(jnp.exp(x - m), axis=1)
    cols = jax.lax.broadcasted_iota(jnp.int32, x.shape, 1)
    xl = jnp.max(jnp.where(cols == lab[:, None], x, -jnp.inf), axis=1)
    loss = m[:, 0] + jnp.log(s) - xl
    loss_ref[pl.ds(i, 1), :] = loss.reshape(1, _BLK)

    @pl.when(i == _G - 1)
    def _finalize():
        losses = loss_ref[...]
        total = jnp.sum(losses)
        bits = jax.lax.bitcast_convert_type(losses, jnp.int32)

        def step(j, t):
            cand = t | jnp.left_shift(jnp.int32(1), 30 - j)
            cnt = jnp.sum(jnp.where(bits >= cand, 1, 0))
            return jnp.where(cnt >= _K, cand, t)

        t = jax.lax.fori_loop(0, 31, step, jnp.int32(0))
        tf = jax.lax.bitcast_convert_type(t, jnp.float32)
        n_gt = jnp.sum(jnp.where(bits > t, 1.0, 0.0))
        sum_gt = jnp.sum(jnp.where(bits > t, losses, 0.0))
        topk_sum = sum_gt + (_K - n_gt) * tf
        result = (total - topk_sum) / _N
        out_ref[...] = jnp.broadcast_to(result, (1, 1))


@jax.jit
def kernel(classes, labels):
    labels3 = labels.astype(jnp.int32).reshape(_G, 1, _BLK)
    out = pl.pallas_call(
        _body,
        grid=(_G,),
        in_specs=[
            pl.BlockSpec((1, 1, _BLK), lambda i: (i, 0, 0)),
            pl.BlockSpec((_BLK, _C), lambda i: (i, 0)),
        ],
        out_specs=pl.BlockSpec((1, 1), lambda i: (0, 0)),
        out_shape=jax.ShapeDtypeStruct((1, 1), jnp.float32),
        scratch_shapes=[pltpu.VMEM((_G, _BLK), jnp.float32)],
    )(labels3, classes)
    return out[0, 0]
